# baseline (speedup 1.0000x reference)
import functools
import math

import jax
import jax.numpy as jnp
from jax import lax
from jax.experimental import pallas as pl
from jax.experimental.pallas import tpu as pltpu
from jax.experimental.pallas import tpu_sc as plsc

D_MODEL = 64
D_PAD = 128
SCALE = math.sqrt(D_MODEL)
NUM_WORKERS = 32
V_CHUNK = 256
CHUNK = 128
NBUF = 4

_params = pltpu.CompilerParams(use_tc_tiling_on_sc=True,
                               needs_layout_passes=False)


def _mesh():
    return plsc.VectorSubcoreMesh(core_axis_name="c", subcore_axis_name="s")


def _sc_pack(table_t, tail, vocab):
    n_total = vocab // V_CHUNK
    v_tail = vocab - n_total * V_CHUNK
    n_per_w = (n_total + NUM_WORKERS - 1) // NUM_WORKERS
    p_chunk = V_CHUNK // 2

    @functools.partial(
        pl.kernel,
        out_type=jax.ShapeDtypeStruct((vocab // 2, D_PAD), jnp.float32),
        mesh=_mesh(),
        scratch_types=[
            pltpu.VMEM((D_MODEL, V_CHUNK), jnp.float32),
            pltpu.VMEM((D_MODEL, V_CHUNK), jnp.float32),
            pltpu.VMEM((p_chunk, D_PAD), jnp.float32),
            pltpu.VMEM((p_chunk, D_PAD), jnp.float32),
            pltpu.VMEM((v_tail, D_MODEL), jnp.float32),
            pltpu.SemaphoreType.DMA,
            pltpu.SemaphoreType.DMA,
            pltpu.SemaphoreType.DMA,
            pltpu.SemaphoreType.DMA,
        ],
        compiler_params=_params,
    )
    def k(tt_hbm, tail_hbm, out_hbm, a0, a1, b0, b1, tl, ga0, ga1, sa0, sa1):
        a = (a0, a1)
        b = (b0, b1)
        gsem = (ga0, ga1)
        ssem = (sa0, sa1)
        wid = lax.axis_index("s") * 2 + lax.axis_index("c")
        base = wid * n_per_w

        def active(i):
            return jnp.logical_and(i < n_per_w, base + i < n_total)

        def start_load(i, buf):
            v0 = (base + i) * V_CHUNK
            pltpu.async_copy(tt_hbm.at[:, pl.ds(v0, V_CHUNK)], a[buf],
                             gsem[buf])

        def wait_load(buf):
            pltpu.make_async_copy(tt_hbm.at[:, pl.ds(0, V_CHUNK)], a[buf],
                                  gsem[buf]).wait()

        def transpose_buf(buf):
            iota = jax.lax.iota(jnp.int32, 16)
            rows_k = []
            colp_k = []
            for k in range(0, V_CHUNK, 16):
                c = iota + k
                rows_k.append(jax.lax.shift_right_logical(c, 1))
                colp_k.append((c & 1) * D_MODEL)

            @plsc.parallel_loop(0, D_MODEL, 1, unroll=4)
            def _(f):
                for k in range(0, V_CHUNK, 16):
                    vals = a[buf][f, pl.ds(k, 16)]
                    plsc.store_scatter(
                        b[buf], [rows_k[k // 16], colp_k[k // 16] + f],
                        vals * SCALE)

        def start_store(i, buf):
            p0 = (base + i) * p_chunk
            pltpu.async_copy(b[buf], out_hbm.at[pl.ds(p0, p_chunk)],
                             ssem[buf])

        def wait_store(buf):
            pltpu.make_async_copy(b[buf], out_hbm.at[pl.ds(0, p_chunk)],
                                  ssem[buf]).wait()

        @pl.when(active(0))
        def _():
            start_load(0, 0)

        def body(i, carry):
            for buf in (0, 1):
                c = 2 * i + buf
                other = 1 - buf

                @pl.when(active(c))
                def _():
                    wait_load(buf)

                    @pl.when(c > 0)
                    def _():
                        wait_store(other)

                    @pl.when(active(c + 1))
                    def _():
                        start_load(c + 1, other)

                    transpose_buf(buf)
                    start_store(c, buf)
            return carry

        lax.fori_loop(0, (n_per_w + 1) // 2, body, 0)
        n_mine = jnp.minimum(n_total - base, n_per_w)

        @pl.when(n_mine > 0)
        def _():
            @pl.when(n_mine % 2 == 1)
            def _():
                wait_store(0)

            @pl.when(n_mine % 2 == 0)
            def _():
                wait_store(1)

        if v_tail:
            @pl.when(wid == 0)
            def _():
                pltpu.sync_copy(tail_hbm, tl)

                def pack_q(q, carry):
                    for h in (0, 1):
                        for j in range(D_MODEL // 16):
                            s = pl.ds(16 * j, 16)
                            b[0][q, pl.ds(h * 64 + 16 * j, 16)] = (
                                tl[2 * q + h, s] * SCALE)
                    return carry
                lax.fori_loop(0, v_tail // 2, pack_q, 0)
                pltpu.sync_copy(
                    b[0].at[pl.ds(0, v_tail // 2)],
                    out_hbm.at[pl.ds((vocab - v_tail) // 2, v_tail // 2)])

    return k(table_t, tail)


def _sc_gather(idx_flat, pair_table, n_idx):
    i_per_w = n_idx // NUM_WORKERS
    n_chunks = i_per_w // CHUNK
    assert n_chunks % NBUF == 0 and CHUNK % 8 == 0
    assert CHUNK % 16 == 0

    @functools.partial(
        pl.kernel,
        out_type=jax.ShapeDtypeStruct((n_idx, D_MODEL), jnp.float32),
        mesh=_mesh(),
        scratch_types=[
            pltpu.VMEM((i_per_w,), jnp.int32),
            pltpu.VMEM((CHUNK, D_PAD), jnp.float32),
            pltpu.VMEM((CHUNK, D_PAD), jnp.float32),
            pltpu.VMEM((CHUNK, D_PAD), jnp.float32),
            pltpu.VMEM((CHUNK, D_PAD), jnp.float32),
            pltpu.VMEM((CHUNK, D_MODEL), jnp.float32),
            pltpu.VMEM((CHUNK, D_MODEL), jnp.float32),
            pltpu.VMEM((CHUNK,), jnp.int32),
            pltpu.VMEM((CHUNK,), jnp.int32),
            pltpu.VMEM((CHUNK,), jnp.int32),
            pltpu.VMEM((CHUNK,), jnp.int32),
            pltpu.SemaphoreType.DMA,
            pltpu.SemaphoreType.DMA,
            pltpu.SemaphoreType.DMA,
            pltpu.SemaphoreType.DMA,
            pltpu.SemaphoreType.DMA,
            pltpu.SemaphoreType.DMA,
        ],
        compiler_params=_params,
    )
    def k(idx_hbm, table_hbm, out_hbm, idx_slab, g0, g1, g2, g3, sb0, sb1,
          p0, p1, p2, p3, gs0, gs1, gs2, gs3, ss0, ss1):
        ga = (g0, g1, g2, g3)
        sb = (sb0, sb1)
        pb = (p0, p1, p2, p3)
        gsem = (gs0, gs1, gs2, gs3)
        ssem = (ss0, ss1)
        wid = lax.axis_index("s") * 2 + lax.axis_index("c")
        base = wid * i_per_w

        pltpu.sync_copy(idx_hbm.at[pl.ds(base, i_per_w)], idx_slab)

        def start_gather(ci, buf):
            def mk(v, carry):
                s = pl.ds(ci * CHUNK + 16 * v, 16)
                pb[buf][pl.ds(16 * v, 16)] = (
                    jax.lax.shift_right_logical(idx_slab[s], 1))
                return carry
            lax.fori_loop(0, CHUNK // 16, mk, 0, unroll=True)
            pltpu.async_copy(table_hbm.at[pb[buf]], ga[buf], gsem[buf])

        def wait_gather(buf):
            pltpu.make_async_copy(table_hbm.at[pb[buf]], ga[buf],
                                  gsem[buf]).wait()

        def compact_buf(ci, gbuf, cbuf):
            def blk(k, carry):
                iv = idx_slab[pl.ds(ci * CHUNK + 16 * k, 16)]
                for r in range(16):
                    off = (iv[r] & 1) * D_MODEL
                    i = 16 * k + r
                    for j in range(D_MODEL // 16):
                        sb[cbuf][i, pl.ds(16 * j, 16)] = (
                            ga[gbuf][i, pl.ds(off + 16 * j, 16)])
                return carry
            lax.fori_loop(0, CHUNK // 16, blk, 0)

        def start_store(ci, cbuf):
            pltpu.async_copy(sb[cbuf],
                             out_hbm.at[pl.ds(base + ci * CHUNK, CHUNK)],
                             ssem[cbuf])

        def wait_store(cbuf):
            pltpu.make_async_copy(sb[cbuf], out_hbm.at[pl.ds(0, CHUNK)],
                                  ssem[cbuf]).wait()

        for p in range(NBUF - 1):
            start_gather(p, p)

        def outer(g, carry):
            for buf in range(NBUF):
                ci = NBUF * g + buf
                nxt = (buf + NBUF - 1) % NBUF
                cbuf = buf % 2
                wait_gather(buf)

                @pl.when(ci + NBUF - 1 < n_chunks)
                def _():
                    start_gather(ci + NBUF - 1, nxt)

                @pl.when(ci >= 2)
                def _():
                    wait_store(cbuf)
                compact_buf(ci, buf, cbuf)
                start_store(ci, cbuf)
            return carry

        lax.fori_loop(0, n_chunks // NBUF, outer, 0)
        wait_store(0)
        wait_store(1)

    return k(idx_flat, pair_table)


def kernel(x, table):
    batch, seq = x.shape
    vocab = table.shape[0]
    idx_flat = x.reshape(batch * seq)
    v_pack = (vocab // V_CHUNK) * V_CHUNK
    pair_table = _sc_pack(table.T, table[v_pack:], vocab)
    out = _sc_gather(idx_flat, pair_table, batch * seq)
    return out.reshape(batch, seq, D_MODEL)


# speedup vs baseline: 1.2832x; 1.2832x over previous
import functools
import math

import jax
import jax.numpy as jnp
from jax import lax
from jax.experimental import pallas as pl
from jax.experimental.pallas import tpu as pltpu
from jax.experimental.pallas import tpu_sc as plsc

D_MODEL = 64
D_PAD = 128
SCALE = math.sqrt(D_MODEL)
NUM_WORKERS = 32
CHUNK = 128
NBUF = 4


def _sc_gather(idx_flat, table_pad, n_idx):
    i_per_w = n_idx // NUM_WORKERS
    n_chunks = i_per_w // CHUNK
    assert n_chunks % NBUF == 0 and CHUNK % 8 == 0
    mesh = plsc.VectorSubcoreMesh(core_axis_name="c", subcore_axis_name="s")

    @functools.partial(
        pl.kernel,
        out_type=jax.ShapeDtypeStruct((n_idx, D_MODEL), jnp.float32),
        mesh=mesh,
        scratch_types=[
            pltpu.VMEM((i_per_w,), jnp.int32),
            pltpu.VMEM((CHUNK, D_PAD), jnp.float32),
            pltpu.VMEM((CHUNK, D_PAD), jnp.float32),
            pltpu.VMEM((CHUNK, D_PAD), jnp.float32),
            pltpu.VMEM((CHUNK, D_PAD), jnp.float32),
            pltpu.VMEM((CHUNK, D_MODEL), jnp.float32),
            pltpu.VMEM((CHUNK, D_MODEL), jnp.float32),
            pltpu.SemaphoreType.DMA,
            pltpu.SemaphoreType.DMA,
            pltpu.SemaphoreType.DMA,
            pltpu.SemaphoreType.DMA,
            pltpu.SemaphoreType.DMA,
            pltpu.SemaphoreType.DMA,
        ],
        compiler_params=pltpu.CompilerParams(use_tc_tiling_on_sc=True),
    )
    def k(idx_hbm, table_hbm, out_hbm, idx_slab, g0, g1, g2, g3, sb0, sb1,
          gs0, gs1, gs2, gs3, ss0, ss1):
        ga = (g0, g1, g2, g3)
        sb = (sb0, sb1)
        gsem = (gs0, gs1, gs2, gs3)
        ssem = (ss0, ss1)
        wid = lax.axis_index("s") * 2 + lax.axis_index("c")
        base = wid * i_per_w

        pltpu.sync_copy(idx_hbm.at[pl.ds(base, i_per_w)], idx_slab)

        def start_gather(ci, buf):
            pltpu.async_copy(
                table_hbm.at[idx_slab.at[pl.ds(ci * CHUNK, CHUNK)]], ga[buf],
                gsem[buf])

        def wait_gather(buf):
            pltpu.make_async_copy(
                table_hbm.at[idx_slab.at[pl.ds(0, CHUNK)]], ga[buf],
                gsem[buf]).wait()

        def scale_buf(gbuf, cbuf):
            def scale_row(i, carry):
                for j in range(D_MODEL // 16):
                    s = pl.ds(j * 16, 16)
                    sb[cbuf][i, s] = ga[gbuf][i, s] * SCALE
                return carry
            lax.fori_loop(0, CHUNK, scale_row, 0, unroll=4)

        def start_store(ci, cbuf):
            pltpu.async_copy(sb[cbuf],
                             out_hbm.at[pl.ds(base + ci * CHUNK, CHUNK)],
                             ssem[cbuf])

        def wait_store(cbuf):
            pltpu.make_async_copy(sb[cbuf], out_hbm.at[pl.ds(0, CHUNK)],
                                  ssem[cbuf]).wait()

        for p in range(NBUF - 1):
            start_gather(p, p)

        def outer(g, carry):
            for buf in range(NBUF):
                ci = NBUF * g + buf
                nxt = (buf + NBUF - 1) % NBUF
                cbuf = buf % 2
                wait_gather(buf)
                @pl.when(ci + NBUF - 1 < n_chunks)
                def _():
                    start_gather(ci + NBUF - 1, nxt)
                @pl.when(ci >= 2)
                def _():
                    wait_store(cbuf)
                scale_buf(buf, cbuf)
                start_store(ci, cbuf)
            return carry

        lax.fori_loop(0, n_chunks // NBUF, outer, 0)
        wait_store(0)
        wait_store(1)

    return k(idx_flat, table_pad)


def kernel(x, table):
    batch, seq = x.shape
    idx_flat = x.reshape(batch * seq)
    table_pad = jnp.pad(table, ((0, 0), (0, D_PAD - D_MODEL)))
    out = _sc_gather(idx_flat, table_pad, batch * seq)
    return out.reshape(batch, seq, D_MODEL)
